# baseline (speedup 1.0000x reference)
import jax
import jax.numpy as jnp
from jax import lax
from jax.experimental import pallas as pl
from jax.experimental.pallas import tpu as pltpu, tpu_sc as plsc

B = 4096
SEQ_LEN = 256
N_BINS = 256
LATENT_DIM = 64

NUM_CORES = 2
NUM_SUBCORES = 16
NW = NUM_CORES * NUM_SUBCORES
BPW = B // NW
RPW = BPW * SEQ_LEN
HALF = 128
BLKB = 8
NBLK = BPW // BLKB
CG = LATENT_DIM // 16


def _body(xs, token_tab, scale_tab, pos_tab, out,
          pos_v, obuf, sbuf, idx_v,
          sem_gt0, sem_gt1, sem_gs0, sem_gs1, sem_o0, sem_o1, sem_i):
    wid = lax.axis_index("s") * NUM_CORES + lax.axis_index("c")
    batch_base = wid * BPW
    sem_gt = (sem_gt0, sem_gt1)
    sem_gs = (sem_gs0, sem_gs1)
    sem_o = (sem_o0, sem_o1)

    pltpu.sync_copy(pos_tab, pos_v)
    pltpu.sync_copy(xs.at[pl.ds(batch_base, BLKB)], idx_v.at[0])

    def issue_tok(h, ls, m, p):
        for j in range(2):
            pltpu.async_copy(token_tab.at[idx_v.at[h, ls, 0, j]],
                             obuf.at[m, pl.ds(j * HALF, HALF)], sem_gt[p])

    def issue_scl(h, ls, p):
        for j in range(2):
            pltpu.async_copy(scale_tab.at[idx_v.at[h, ls, 1, j]],
                             sbuf.at[p, pl.ds(j * HALF, HALF)], sem_gs[p])

    def blk_body(blk, _):
        h = lax.rem(blk, 2)

        @pl.when(blk > 0)
        def _wait_idx():
            pltpu.make_async_copy(xs.at[pl.ds(0, BLKB)], idx_v.at[h], sem_i).wait()

        @pl.when(blk + 1 < NBLK)
        def _prefetch_idx():
            nxt = batch_base + (blk + 1) * BLKB
            pltpu.async_copy(xs.at[pl.ds(nxt, BLKB)], idx_v.at[1 - h], sem_i)

        for ls0 in (0, 1):
            issue_tok(h, ls0, ls0, ls0)
            issue_scl(h, ls0, ls0)

        def q_body(q, _):
            for m in range(4):
                p = m % 2
                ls = q * 4 + m
                g = blk * BLKB + ls
                pltpu.make_async_copy(token_tab.at[pl.ds(0, SEQ_LEN)],
                                      obuf.at[m], sem_gt[p]).wait()
                pltpu.make_async_copy(scale_tab.at[pl.ds(0, SEQ_LEN)],
                                      sbuf.at[p], sem_gs[p]).wait()

                @pl.when(g >= 2)
                def _drain_out():
                    pltpu.make_async_copy(token_tab.at[pl.ds(0, SEQ_LEN)],
                                          obuf.at[(m + 2) % 4], sem_o[p]).wait()

                @pl.when(ls + 2 < BLKB)
                def _prefetch_tok():
                    issue_tok(h, ls + 2, (m + 2) % 4, p)

                def row_body(r, _):
                    for c in range(CG):
                        sl = pl.ds(c * 16, 16)
                        v = sbuf[p, r, sl] + pos_v[r, sl]
                        plsc.addupdate(obuf.at[m, r, sl], v)
                    return 0

                lax.fori_loop(0, SEQ_LEN, row_body, 0)

                pltpu.async_copy(obuf.at[m],
                                 out.at[pl.ds((batch_base + g) * SEQ_LEN, SEQ_LEN)],
                                 sem_o[p])

                @pl.when(ls + 2 < BLKB)
                def _prefetch_scl():
                    issue_scl(h, ls + 2, p)
            return 0

        lax.fori_loop(0, BLKB // 4, q_body, 0)
        return 0

    lax.fori_loop(0, NBLK, blk_body, 0)

    pltpu.make_async_copy(token_tab.at[pl.ds(0, SEQ_LEN)], obuf.at[0], sem_o0).wait()
    pltpu.make_async_copy(token_tab.at[pl.ds(0, SEQ_LEN)], obuf.at[1], sem_o1).wait()


@jax.jit
def _run(xs, token_table, scale_table, pos_table):
    mesh = plsc.VectorSubcoreMesh(core_axis_name="c", subcore_axis_name="s")
    kfn = pl.kernel(
        _body,
        out_type=jax.ShapeDtypeStruct((B * SEQ_LEN, LATENT_DIM), jnp.float32),
        mesh=mesh,
        compiler_params=pltpu.CompilerParams(use_tc_tiling_on_sc=False),
        scratch_types=[
            pltpu.VMEM((SEQ_LEN, LATENT_DIM), jnp.float32),
            pltpu.VMEM((4, SEQ_LEN, LATENT_DIM), jnp.float32),
            pltpu.VMEM((2, SEQ_LEN, LATENT_DIM), jnp.float32),
            pltpu.VMEM((2, BLKB, 2, 2, HALF), jnp.int32),
            pltpu.SemaphoreType.DMA,
            pltpu.SemaphoreType.DMA,
            pltpu.SemaphoreType.DMA,
            pltpu.SemaphoreType.DMA,
            pltpu.SemaphoreType.DMA,
            pltpu.SemaphoreType.DMA,
            pltpu.SemaphoreType.DMA,
        ],
    )
    return kfn(xs, token_table, scale_table, pos_table)


def kernel(x, token_table, scale_table, pos_table):
    xs = x.reshape(B, 2, 2, HALF)
    out = _run(xs, token_table, scale_table, pos_table)
    return out.reshape(B, SEQ_LEN, LATENT_DIM)


# speedup vs baseline: 1.5813x; 1.5813x over previous
import jax
import jax.numpy as jnp
from jax import lax
from jax.experimental import pallas as pl
from jax.experimental.pallas import tpu as pltpu, tpu_sc as plsc

B = 4096
SEQ_LEN = 256
N_BINS = 256
LATENT_DIM = 64

NUM_CORES = 2
NUM_SUBCORES = 16
NW = NUM_CORES * NUM_SUBCORES
BPW = B // NW
RPW = BPW * SEQ_LEN
HALF = 128
BLKB = 8
NBLK = BPW // BLKB
CG = LATENT_DIM // 16


def _body(xs, token_tab, scale_tab, pos_tab, out,
          pos_v, obuf, sbuf, idx_v, tok_sh, scl_sh,
          sem_gt0, sem_gt1, sem_gs0, sem_gs1, sem_o0, sem_o1, sem_i):
    sid = lax.axis_index("s")
    wid = sid * NUM_CORES + lax.axis_index("c")
    batch_base = wid * BPW
    sem_gt = (sem_gt0, sem_gt1)
    sem_gs = (sem_gs0, sem_gs1)
    sem_o = (sem_o0, sem_o1)

    @pl.when(sid == 0)
    def _stage_tables():
        pltpu.sync_copy(token_tab, tok_sh)
        pltpu.sync_copy(scale_tab, scl_sh)

    pltpu.sync_copy(pos_tab, pos_v)
    pltpu.sync_copy(xs.at[pl.ds(batch_base, BLKB)], idx_v.at[0])
    plsc.subcore_barrier()

    def issue_tok(h, ls, m, p):
        for j in range(2):
            pltpu.async_copy(tok_sh.at[idx_v.at[h, ls, 0, j]],
                             obuf.at[m, pl.ds(j * HALF, HALF)], sem_gt[p])

    def issue_scl(h, ls, p):
        for j in range(2):
            pltpu.async_copy(scl_sh.at[idx_v.at[h, ls, 1, j]],
                             sbuf.at[p, pl.ds(j * HALF, HALF)], sem_gs[p])

    def blk_body(blk, _):
        h = lax.rem(blk, 2)

        @pl.when(blk > 0)
        def _wait_idx():
            pltpu.make_async_copy(xs.at[pl.ds(0, BLKB)], idx_v.at[h], sem_i).wait()

        @pl.when(blk + 1 < NBLK)
        def _prefetch_idx():
            nxt = batch_base + (blk + 1) * BLKB
            pltpu.async_copy(xs.at[pl.ds(nxt, BLKB)], idx_v.at[1 - h], sem_i)

        for ls0 in (0, 1):
            issue_tok(h, ls0, ls0, ls0)
            issue_scl(h, ls0, ls0)

        def q_body(q, _):
            for m in range(4):
                p = m % 2
                ls = q * 4 + m
                g = blk * BLKB + ls
                pltpu.make_async_copy(token_tab.at[pl.ds(0, SEQ_LEN)],
                                      obuf.at[m], sem_gt[p]).wait()
                pltpu.make_async_copy(scale_tab.at[pl.ds(0, SEQ_LEN)],
                                      sbuf.at[p], sem_gs[p]).wait()

                @pl.when(g >= 2)
                def _drain_out():
                    pltpu.make_async_copy(token_tab.at[pl.ds(0, SEQ_LEN)],
                                          obuf.at[(m + 2) % 4], sem_o[p]).wait()

                @pl.when(ls + 2 < BLKB)
                def _prefetch_tok():
                    issue_tok(h, ls + 2, (m + 2) % 4, p)

                def row_body(r, _):
                    for c in range(CG):
                        sl = pl.ds(c * 16, 16)
                        v = sbuf[p, r, sl] + pos_v[r, sl]
                        plsc.addupdate(obuf.at[m, r, sl], v)
                    return 0

                lax.fori_loop(0, SEQ_LEN, row_body, 0)

                pltpu.async_copy(obuf.at[m],
                                 out.at[pl.ds((batch_base + g) * SEQ_LEN, SEQ_LEN)],
                                 sem_o[p])

                @pl.when(ls + 2 < BLKB)
                def _prefetch_scl():
                    issue_scl(h, ls + 2, p)
            return 0

        lax.fori_loop(0, BLKB // 4, q_body, 0)
        return 0

    lax.fori_loop(0, NBLK, blk_body, 0)

    pltpu.make_async_copy(token_tab.at[pl.ds(0, SEQ_LEN)], obuf.at[0], sem_o0).wait()
    pltpu.make_async_copy(token_tab.at[pl.ds(0, SEQ_LEN)], obuf.at[1], sem_o1).wait()


@jax.jit
def _run(xs, token_table, scale_table, pos_table):
    mesh = plsc.VectorSubcoreMesh(core_axis_name="c", subcore_axis_name="s")
    kfn = pl.kernel(
        _body,
        out_type=jax.ShapeDtypeStruct((B * SEQ_LEN, LATENT_DIM), jnp.float32),
        mesh=mesh,
        compiler_params=pltpu.CompilerParams(use_tc_tiling_on_sc=False),
        scratch_types=[
            pltpu.VMEM((SEQ_LEN, LATENT_DIM), jnp.float32),
            pltpu.VMEM((4, SEQ_LEN, LATENT_DIM), jnp.float32),
            pltpu.VMEM((2, SEQ_LEN, LATENT_DIM), jnp.float32),
            pltpu.VMEM((2, BLKB, 2, 2, HALF), jnp.int32),
            pltpu.VMEM_SHARED((N_BINS, LATENT_DIM), jnp.float32),
            pltpu.VMEM_SHARED((N_BINS, LATENT_DIM), jnp.float32),
            pltpu.SemaphoreType.DMA,
            pltpu.SemaphoreType.DMA,
            pltpu.SemaphoreType.DMA,
            pltpu.SemaphoreType.DMA,
            pltpu.SemaphoreType.DMA,
            pltpu.SemaphoreType.DMA,
            pltpu.SemaphoreType.DMA,
        ],
    )
    return kfn(xs, token_table, scale_table, pos_table)


def kernel(x, token_table, scale_table, pos_table):
    xs = x.reshape(B, 2, 2, HALF)
    out = _run(xs, token_table, scale_table, pos_table)
    return out.reshape(B, SEQ_LEN, LATENT_DIM)


# speedup vs baseline: 1.5830x; 1.0011x over previous
import jax
import jax.numpy as jnp
from jax import lax
from jax.experimental import pallas as pl
from jax.experimental.pallas import tpu as pltpu, tpu_sc as plsc

B = 4096
SEQ_LEN = 256
N_BINS = 256
LATENT_DIM = 64

NUM_CORES = 2
NUM_SUBCORES = 16
NW = NUM_CORES * NUM_SUBCORES
BPW = B // NW
RPW = BPW * SEQ_LEN
HALF = 128
BLKB = 8
NBLK = BPW // BLKB
CG = LATENT_DIM // 16


def _body(xs, token_tab, scale_tab, pos_tab, out,
          pos_v, obuf, sbuf, idx_v, tok_sh, scl_sh,
          sem_gt0, sem_gt1, sem_gs0, sem_gs1, sem_o0, sem_o1, sem_i):
    sid = lax.axis_index("s")
    wid = sid * NUM_CORES + lax.axis_index("c")
    batch_base = wid * BPW
    sem_gt = (sem_gt0, sem_gt1)
    sem_gs = (sem_gs0, sem_gs1)
    sem_o = (sem_o0, sem_o1)

    @pl.when(sid == 0)
    def _stage_tables():
        pltpu.sync_copy(token_tab, tok_sh)
        pltpu.sync_copy(scale_tab, scl_sh)

    pltpu.sync_copy(pos_tab, pos_v)
    pltpu.sync_copy(xs.at[pl.ds(batch_base, BLKB)], idx_v.at[0])
    plsc.subcore_barrier()

    def issue_tok(h, ls, m, p):
        for j in range(2):
            pltpu.async_copy(tok_sh.at[idx_v.at[h, ls, 0, j]],
                             obuf.at[m, pl.ds(j * HALF, HALF)], sem_gt[p])

    def issue_scl(h, ls, p):
        for j in range(2):
            pltpu.async_copy(scl_sh.at[idx_v.at[h, ls, 1, j]],
                             sbuf.at[p, pl.ds(j * HALF, HALF)], sem_gs[p])

    def blk_body(blk, _):
        h = lax.rem(blk, 2)

        @pl.when(blk > 0)
        def _wait_idx():
            pltpu.make_async_copy(xs.at[pl.ds(0, BLKB)], idx_v.at[h], sem_i).wait()

        @pl.when(blk + 1 < NBLK)
        def _prefetch_idx():
            nxt = batch_base + (blk + 1) * BLKB
            pltpu.async_copy(xs.at[pl.ds(nxt, BLKB)], idx_v.at[1 - h], sem_i)

        for ls0 in (0, 1):
            issue_tok(h, ls0, ls0, ls0)
            issue_scl(h, ls0, ls0)

        def q_body(q, _):
            for m in range(4):
                p = m % 2
                ls = q * 4 + m
                g = blk * BLKB + ls
                pltpu.make_async_copy(token_tab.at[pl.ds(0, SEQ_LEN)],
                                      obuf.at[m], sem_gt[p]).wait()
                pltpu.make_async_copy(scale_tab.at[pl.ds(0, SEQ_LEN)],
                                      sbuf.at[p], sem_gs[p]).wait()

                @pl.when(g >= 2)
                def _drain_out():
                    pltpu.make_async_copy(token_tab.at[pl.ds(0, SEQ_LEN)],
                                          obuf.at[(m + 2) % 4], sem_o[p]).wait()

                @pl.when(ls + 2 < BLKB)
                def _prefetch_tok():
                    issue_tok(h, ls + 2, (m + 2) % 4, p)

                def row_body(r, _):
                    for c in range(CG):
                        sl = pl.ds(c * 16, 16)
                        v = sbuf[p, r, sl] + pos_v[r, sl]
                        plsc.addupdate(obuf.at[m, r, sl], v)
                    return 0

                lax.fori_loop(0, SEQ_LEN, row_body, 0)

                pltpu.async_copy(obuf.at[m], out.at[batch_base + g], sem_o[p])

                @pl.when(ls + 2 < BLKB)
                def _prefetch_scl():
                    issue_scl(h, ls + 2, p)
            return 0

        lax.fori_loop(0, BLKB // 4, q_body, 0)
        return 0

    lax.fori_loop(0, NBLK, blk_body, 0)

    pltpu.make_async_copy(token_tab.at[pl.ds(0, SEQ_LEN)], obuf.at[0], sem_o0).wait()
    pltpu.make_async_copy(token_tab.at[pl.ds(0, SEQ_LEN)], obuf.at[1], sem_o1).wait()


@jax.jit
def _run(xs, token_table, scale_table, pos_table):
    mesh = plsc.VectorSubcoreMesh(core_axis_name="c", subcore_axis_name="s")
    kfn = pl.kernel(
        _body,
        out_type=jax.ShapeDtypeStruct((B, SEQ_LEN, LATENT_DIM), jnp.float32),
        mesh=mesh,
        compiler_params=pltpu.CompilerParams(use_tc_tiling_on_sc=False),
        scratch_types=[
            pltpu.VMEM((SEQ_LEN, LATENT_DIM), jnp.float32),
            pltpu.VMEM((4, SEQ_LEN, LATENT_DIM), jnp.float32),
            pltpu.VMEM((2, SEQ_LEN, LATENT_DIM), jnp.float32),
            pltpu.VMEM((2, BLKB, 2, 2, HALF), jnp.int32),
            pltpu.VMEM_SHARED((N_BINS, LATENT_DIM), jnp.float32),
            pltpu.VMEM_SHARED((N_BINS, LATENT_DIM), jnp.float32),
            pltpu.SemaphoreType.DMA,
            pltpu.SemaphoreType.DMA,
            pltpu.SemaphoreType.DMA,
            pltpu.SemaphoreType.DMA,
            pltpu.SemaphoreType.DMA,
            pltpu.SemaphoreType.DMA,
            pltpu.SemaphoreType.DMA,
        ],
    )
    return kfn(xs, token_table, scale_table, pos_table)


def kernel(x, token_table, scale_table, pos_table):
    xs = x.reshape(B, 2, 2, HALF)
    return _run(xs, token_table, scale_table, pos_table)
